# baseline (device time: 91927 ns/iter reference)
import jax
import jax.numpy as jnp
from jax import lax
from jax.experimental import pallas as pl
from jax.experimental.pallas import tpu as pltpu

N_DEV = 16
N_RING = 6
B_PER = 2
SQ = 128
SKV = 128
HQ = 64
HQ_PER = 4
DH = 64
D_MODEL = 512
HD_PER = HQ_PER * DH
TOK = B_PER * SQ
SCALE = 0.125

_OFFS = [0, -1, 1, -4, 4, 8, -2, 2, -3, 3, -5, 5, -6, 6, -7, 7]

_RING_D = [1, 2, 3, 5, 6, 7]


def kernel(x, Wq, K_ext, V_ext, Wo):
    my = lax.axis_index("i")

    w_cat = jnp.concatenate([Wq * SCALE, Wo.T], axis=1).astype(jnp.bfloat16)

    k_loc = lax.dynamic_slice(K_ext, (my * B_PER, 0, 0, 0), (B_PER, SKV, HQ, DH))
    v_loc = lax.dynamic_slice(V_ext, (my * B_PER, 0, 0, 0), (B_PER, SKV, HQ, DH))
    perm = (my + jnp.array(_OFFS)) % N_DEV
    k_arr = k_loc.reshape(B_PER, SKV, N_DEV, HQ_PER, DH).transpose(2, 0, 3, 1, 4)[perm]
    v_arr = v_loc.reshape(B_PER, SKV, N_DEV, HQ_PER, DH).transpose(2, 0, 3, 1, 4)[perm]
    k_arr = k_arr.astype(jnp.bfloat16)
    v_arr = v_arr.astype(jnp.bfloat16)

    def body(x_ref, w_ref, k_ref, v_ref, out_ref,
             cw_ref, ccw_ref, z_ref,
             cw_send, cw_recv, ccw_send, ccw_recv, z_send, z_recv):
        me = lax.axis_index("i")
        left = (me + N_DEV - 1) % N_DEV
        right = (me + 1) % N_DEV
        up4 = (me + 4) % N_DEV
        dn4 = (me + N_DEV - 4) % N_DEV
        up8 = (me + 8) % N_DEV

        barrier_sem = pltpu.get_barrier_semaphore()
        for nbr in (left, right, up4, dn4, up8):
            pl.semaphore_signal(
                barrier_sem, inc=1,
                device_id=(nbr,), device_id_type=pl.DeviceIdType.MESH,
            )
        pl.semaphore_wait(barrier_sem, 5)

        x2 = x_ref[...].reshape(TOK, D_MODEL).astype(jnp.bfloat16)

        def compute(t, w):
            wq = w[:, :HD_PER]
            woT = w[:, HD_PER:]
            q = jnp.dot(x2, wq, preferred_element_type=jnp.float32)
            q4t = (
                q.reshape(B_PER, SQ, HQ_PER, DH)
                .transpose(0, 2, 1, 3)
                .reshape(B_PER * HQ_PER, SQ, DH)
                .astype(jnp.bfloat16)
            )
            k5 = k_ref[t].reshape(B_PER * HQ_PER, SKV, DH)
            v5 = v_ref[t].reshape(B_PER * HQ_PER, SKV, DH)
            s = lax.dot_general(
                q4t, k5, (((2,), (2,)), ((0,), (0,))),
                preferred_element_type=jnp.float32,
            )
            e = jnp.exp(s)
            r = 1.0 / jnp.sum(e, axis=-1, keepdims=True)
            ctx = lax.dot_general(
                e.astype(jnp.bfloat16), v5, (((2,), (1,)), ((0,), (0,))),
                preferred_element_type=jnp.float32,
            )
            ctx2 = (
                (ctx * r).reshape(B_PER, HQ_PER, SQ, DH)
                .transpose(0, 2, 1, 3)
                .reshape(TOK, HD_PER).astype(jnp.bfloat16)
            )
            return lax.dot_general(
                ctx2, woT, (((1,), (1,)), ((), ())),
                preferred_element_type=jnp.float32,
            )

        HALF = D_MODEL // 2

        def mk_half(src_ref, src_slot, dst_ref, slot, half, send, recv, dst_dev):
            rows = pl.ds(half * HALF, HALF)
            src = (src_ref.at[rows] if src_slot is None
                   else src_ref.at[src_slot, rows])
            return pltpu.make_async_remote_copy(
                src_ref=src,
                dst_ref=dst_ref.at[slot, rows],
                send_sem=send.at[slot, half],
                recv_sem=recv.at[slot, half],
                device_id=(dst_dev,),
                device_id_type=pl.DeviceIdType.MESH,
            )

        def mk_z(slot, dst_dev):
            return pltpu.make_async_remote_copy(
                src_ref=w_ref,
                dst_ref=z_ref.at[slot],
                send_sem=z_send.at[slot],
                recv_sem=z_recv.at[slot],
                device_id=(dst_dev,),
                device_id_type=pl.DeviceIdType.MESH,
            )

        sends = []

        def issue_ring(direction, src_ref, src_slot, slot):
            dst = right if direction == 0 else left
            send = cw_send if direction == 0 else ccw_send
            recv = cw_recv if direction == 0 else ccw_recv
            dst_ref = cw_ref if direction == 0 else ccw_ref
            pair = []
            for half in (0, 1):
                r = mk_half(src_ref, src_slot, dst_ref, slot, half, send, recv, dst)
                r.start()
                sends.append(r)
                pair.append(r)
            return pair

        cw_p = {0: issue_ring(0, w_ref, None, 0)}
        ccw_p = {0: issue_ring(1, w_ref, None, 0)}
        z_w = []
        for slot, dst in ((0, up4), (1, dn4), (2, up8)):
            r = mk_z(slot, dst)
            r.start()
            sends.append(r)
            z_w.append(r)

        acc = compute(0, w_ref[...])

        for h in (0, 1):
            cw_p[0][h].wait_recv()
        cw_p[1] = issue_ring(0, cw_ref, 0, 1)
        for h in (0, 1):
            ccw_p[0][h].wait_recv()
        ccw_p[1] = issue_ring(1, ccw_ref, 0, 1)
        acc = acc + compute(1, cw_ref[0])
        acc = acc + compute(2, ccw_ref[0])

        z_w[0].wait_recv()
        cw_p[3] = issue_ring(0, z_ref, 0, 3)
        z_w[1].wait_recv()
        ccw_p[3] = issue_ring(1, z_ref, 1, 3)
        acc = acc + compute(3, z_ref[0])
        acc = acc + compute(4, z_ref[1])
        z_w[2].wait_recv()
        acc = acc + compute(5, z_ref[2])

        for k in range(1, N_RING):
            for h in (0, 1):
                cw_p[k][h].wait_recv()
            if k in (1, 3, 4):
                cw_p[k + 1] = issue_ring(0, cw_ref, k, k + 1)
            for h in (0, 1):
                ccw_p[k][h].wait_recv()
            if k in (1, 3, 4):
                ccw_p[k + 1] = issue_ring(1, ccw_ref, k, k + 1)
            acc = acc + compute(6 + 2 * (k - 1), cw_ref[k])
            acc = acc + compute(7 + 2 * (k - 1), ccw_ref[k])

        for r in sends:
            r.wait_send()

        out_ref[...] = acc.reshape(B_PER, SQ, D_MODEL)

    return pl.pallas_call(
        body,
        out_shape=jax.ShapeDtypeStruct((B_PER, SQ, D_MODEL), jnp.float32),
        in_specs=[
            pl.BlockSpec(memory_space=pltpu.VMEM),
            pl.BlockSpec(memory_space=pltpu.VMEM),
            pl.BlockSpec(memory_space=pltpu.VMEM),
            pl.BlockSpec(memory_space=pltpu.VMEM),
        ],
        out_specs=pl.BlockSpec(memory_space=pltpu.VMEM),
        scratch_shapes=[
            pltpu.VMEM((N_RING, D_MODEL, 2 * HD_PER), jnp.bfloat16),
            pltpu.VMEM((N_RING, D_MODEL, 2 * HD_PER), jnp.bfloat16),
            pltpu.VMEM((3, D_MODEL, 2 * HD_PER), jnp.bfloat16),
            pltpu.SemaphoreType.DMA((N_RING, 2)),
            pltpu.SemaphoreType.DMA((N_RING, 2)),
            pltpu.SemaphoreType.DMA((N_RING, 2)),
            pltpu.SemaphoreType.DMA((N_RING, 2)),
            pltpu.SemaphoreType.DMA((3,)),
            pltpu.SemaphoreType.DMA((3,)),
        ],
        compiler_params=pltpu.CompilerParams(collective_id=0),
    )(x, w_cat, k_arr, v_arr)


# device time: 65628 ns/iter; 1.4007x vs baseline; 1.4007x over previous
import jax
import jax.numpy as jnp
from jax import lax
from jax.experimental import pallas as pl
from jax.experimental.pallas import tpu as pltpu

N_DEV = 16
N_CW = 8
N_CCW = 7
B_PER = 2
SQ = 128
SKV = 128
HQ = 64
HQ_PER = 4
DH = 64
D_MODEL = 512
HD_PER = HQ_PER * DH
TOK = B_PER * SQ
SCALE = 0.125

_OFFS = [0]
for _r in range(1, 8):
    _OFFS += [-_r, _r]
_OFFS.append(-8)


def kernel(x, Wq, K_ext, V_ext, Wo):
    my = lax.axis_index("i")

    w_cat = jnp.concatenate([Wq * SCALE, Wo.T], axis=1).astype(jnp.bfloat16)

    k_loc = lax.dynamic_slice(K_ext, (my * B_PER, 0, 0, 0), (B_PER, SKV, HQ, DH))
    v_loc = lax.dynamic_slice(V_ext, (my * B_PER, 0, 0, 0), (B_PER, SKV, HQ, DH))
    perm = (my + jnp.array(_OFFS)) % N_DEV
    k_arr = k_loc.reshape(B_PER, SKV, N_DEV, HQ_PER, DH).transpose(2, 0, 3, 1, 4)[perm]
    v_arr = v_loc.reshape(B_PER, SKV, N_DEV, HQ_PER, DH).transpose(2, 0, 3, 1, 4)[perm]
    k_arr = k_arr.astype(jnp.bfloat16)
    v_arr = v_arr.astype(jnp.bfloat16)

    def body(x_ref, w_ref, k_ref, v_ref, out_ref,
             cw_ref, ccw_ref, cw_send, cw_recv, ccw_send, ccw_recv):
        me = lax.axis_index("i")
        left = (me + N_DEV - 1) % N_DEV
        right = (me + 1) % N_DEV

        barrier_sem = pltpu.get_barrier_semaphore()
        for nbr in (left, right):
            pl.semaphore_signal(
                barrier_sem, inc=1,
                device_id=(nbr,), device_id_type=pl.DeviceIdType.MESH,
            )
        pl.semaphore_wait(barrier_sem, 2)

        x2 = x_ref[...].reshape(TOK, D_MODEL).astype(jnp.bfloat16)

        def compute(t, w):
            wq = w[:, :HD_PER]
            woT = w[:, HD_PER:]
            q = jnp.dot(x2, wq, preferred_element_type=jnp.float32)
            q4t = (
                q.reshape(B_PER, SQ, HQ_PER, DH)
                .transpose(0, 2, 1, 3)
                .reshape(B_PER * HQ_PER, SQ, DH)
                .astype(jnp.bfloat16)
            )
            k5 = k_ref[t].reshape(B_PER * HQ_PER, SKV, DH)
            v5 = v_ref[t].reshape(B_PER * HQ_PER, SKV, DH)
            s = lax.dot_general(
                q4t, k5, (((2,), (2,)), ((0,), (0,))),
                preferred_element_type=jnp.float32,
            )
            e = jnp.exp(s)
            r = 1.0 / jnp.sum(e, axis=-1, keepdims=True)
            ctx = lax.dot_general(
                e.astype(jnp.bfloat16), v5, (((2,), (1,)), ((0,), (0,))),
                preferred_element_type=jnp.float32,
            )
            ctx2 = (
                (ctx * r).reshape(B_PER, HQ_PER, SQ, DH)
                .transpose(0, 2, 1, 3)
                .reshape(TOK, HD_PER).astype(jnp.bfloat16)
            )
            return lax.dot_general(
                ctx2, woT, (((1,), (1,)), ((), ())),
                preferred_element_type=jnp.float32,
            )

        HALF = D_MODEL // 2

        def mk(src_ref, src_slot, dst_ref, slot, half, send, recv, dst_dev):
            rows = pl.ds(half * HALF, HALF)
            src = (src_ref.at[rows] if src_slot is None
                   else src_ref.at[src_slot, rows])
            return pltpu.make_async_remote_copy(
                src_ref=src,
                dst_ref=dst_ref.at[slot, rows],
                send_sem=send.at[slot, half],
                recv_sem=recv.at[slot, half],
                device_id=(dst_dev,),
                device_id_type=pl.DeviceIdType.MESH,
            )

        sends = []
        cw_p = {0: []}
        ccw_p = {0: []}
        for half in (0, 1):
            for pairs, dref, ssem, rsem, dst in (
                (cw_p, cw_ref, cw_send, cw_recv, right),
                (ccw_p, ccw_ref, ccw_send, ccw_recv, left),
            ):
                rr = mk(w_ref, None, dref, 0, half, ssem, rsem, dst)
                rr.start()
                sends.append(rr)
                pairs[0].append(rr)

        acc = compute(0, w_ref[...])

        for r in range(1, 8):
            for half in (0, 1):
                cw_p[r - 1][half].wait_recv()
                if r < 7 or half == 0:
                    nxt = mk(cw_ref, r - 1, cw_ref, r, half, cw_send, cw_recv, right)
                    nxt.start()
                    sends.append(nxt)
                    cw_p.setdefault(r, []).append(nxt)
                ccw_p[r - 1][half].wait_recv()
                if r < 7 or half == 1:
                    nxt = mk(ccw_ref, r - 1, ccw_ref, r, half,
                             ccw_send, ccw_recv, left)
                    nxt.start()
                    sends.append(nxt)
                    ccw_p.setdefault(r, []).append(nxt)
            acc = acc + compute(2 * r - 1, cw_ref[r - 1])
            acc = acc + compute(2 * r, ccw_ref[r - 1])

        cw_p[7][0].wait_recv()
        ccw_p[7][0].wait_recv()
        w15 = jnp.concatenate([cw_ref[7][:HALF], ccw_ref[7][HALF:]], axis=0)
        acc = acc + compute(15, w15)

        for rdma in sends:
            rdma.wait_send()

        out_ref[...] = acc.reshape(B_PER, SQ, D_MODEL)

    return pl.pallas_call(
        body,
        out_shape=jax.ShapeDtypeStruct((B_PER, SQ, D_MODEL), jnp.float32),
        in_specs=[
            pl.BlockSpec(memory_space=pltpu.VMEM),
            pl.BlockSpec(memory_space=pltpu.VMEM),
            pl.BlockSpec(memory_space=pltpu.VMEM),
            pl.BlockSpec(memory_space=pltpu.VMEM),
        ],
        out_specs=pl.BlockSpec(memory_space=pltpu.VMEM),
        scratch_shapes=[
            pltpu.VMEM((N_CW, D_MODEL, 2 * HD_PER), jnp.bfloat16),
            pltpu.VMEM((N_CW, D_MODEL, 2 * HD_PER), jnp.bfloat16),
            pltpu.SemaphoreType.DMA((N_CW, 2)),
            pltpu.SemaphoreType.DMA((N_CW, 2)),
            pltpu.SemaphoreType.DMA((N_CW, 2)),
            pltpu.SemaphoreType.DMA((N_CW, 2)),
        ],
        compiler_params=pltpu.CompilerParams(collective_id=0),
    )(x, w_cat, k_arr, v_arr)
